# baseline (speedup 1.0000x reference)
import functools

import jax
import jax.numpy as jnp
from jax import lax
from jax.experimental import pallas as pl
from jax.experimental.pallas import tpu as pltpu
from jax.experimental.pallas import tpu_sc as plsc

_B, _F, _C, _R = 512, 2000, 8, 4
_T = 2048
_L = 16
_NC, _NS = 2, 16
_NW = _NC * _NS

_NIN = _F * _C * 4
_NOUT = _B * _C * 16
_W = 512
_NTT = _T // 128
_QS = 32


def _make_packing_kernel(interpret=False):
    mesh = plsc.VectorSubcoreMesh(
        core_axis_name="c", subcore_axis_name="s",
        num_cores=_NC, num_subcores=_NS)

    @functools.partial(
        pl.kernel,
        out_type=jax.ShapeDtypeStruct((_NOUT, _W), jnp.float32),
        mesh=mesh,
        scratch_types=[
            pltpu.VMEM((_T,), jnp.int32),
            pltpu.VMEM((4, _QS), jnp.int32),
            pltpu.VMEM((2, 128), jnp.int32),
            pltpu.VMEM((2 * _QS + 1, _W), jnp.float32),
            pltpu.VMEM((128, _W), jnp.float32),
            pltpu.SemaphoreType.DMA,
            pltpu.SemaphoreType.DMA,
            pltpu.SemaphoreType.DMA,
        ],
        interpret=interpret,
        compiler_params=pltpu.CompilerParams(
            needs_layout_passes=False, use_tc_tiling_on_sc=False),
    )
    def packing(lv_hbm, perm_hbm, ov_hbm,
                perm_v, gidx_v, sidx_v, inb_v, outb_v, gsem0, gsem1, wsem):
        wid = lax.axis_index("s") * _NC + lax.axis_index("c")
        c = wid // 4
        bt = wid % 4
        pltpu.sync_copy(perm_hbm, perm_v)

        lane = lax.iota(jnp.int32, _L)
        zeros = jnp.zeros((_L,), jnp.float32)
        gsems = (gsem0, gsem1)

        for q in range(_W // _L):
            inb_v[2 * _QS, pl.ds(q * _L, _L)] = zeros

        obase = bt * 128 * _C * 16 + c * 16

        def build_gidx(tt, q, row):
            for u in range(_QS // _L):
                pv = perm_v[pl.ds(tt * 128 + q * _QS + u * _L, _L)]
                gidx_v[row, pl.ds(u * _L, _L)] = (
                    jnp.minimum(pv, _F - 1) * (_C * 4) + (c * 4 + bt))

        def fire_gather(row, half):
            pltpu.async_copy(
                lv_hbm.at[gidx_v.at[row]],
                inb_v.at[pl.ds(half * _QS, _QS)], gsems[half])

        def wait_gather(row, half):
            pltpu.make_async_copy(
                lv_hbm.at[gidx_v.at[row]],
                inb_v.at[pl.ds(half * _QS, _QS)], gsems[half]).wait()

        def drain_scatter(par):
            pltpu.make_async_copy(
                outb_v, ov_hbm.at[sidx_v.at[par]], wsem).wait()

        build_gidx(0, 0, 0)
        fire_gather(0, 0)

        def tt_body(tt, carry):
            for q in range(128 // _L):
                sidx_v[tt % 2, pl.ds(q * _L, _L)] = (
                    (lane + q * _L) * (_C * 16) + (obase + tt))

            for q in range(4):
                half = q % 2
                if q < 3:
                    build_gidx(tt, q + 1, q + 1)
                    fire_gather(q + 1, (q + 1) % 2)
                else:
                    @pl.when(tt != _NTT - 1)
                    def _():
                        build_gidx(tt + 1, 0, 0)
                        fire_gather(0, 0)

                wait_gather(q, half)

                if q == 0:
                    @pl.when(tt != 0)
                    def _():
                        drain_scatter((tt + 1) % 2)

                fi = []
                for tlc in range(_QS // _L):
                    pv = perm_v[pl.ds(tt * 128 + q * _QS + tlc * _L, _L)]
                    fi.append(jnp.where(pv < _F,
                                        lane + (half * _QS + tlc * _L),
                                        2 * _QS))

                ones = jnp.ones((_L,), jnp.int32)
                col0 = tuple(jnp.full((_L,), r * 128, jnp.int32)
                             for r in range(_R))

                @plsc.parallel_loop(0, 128, unroll=8, carry=col0)
                def j_body(j, cols):
                    for r in range(_R):
                        for tlc in range(_QS // _L):
                            v = plsc.load_gather(inb_v, [fi[tlc], cols[r]])
                            outb_v[j, pl.ds(r * 128 + q * _QS + tlc * _L,
                                            _L)] = v
                    return tuple(cv + ones for cv in cols)

            pltpu.async_copy(outb_v, ov_hbm.at[sidx_v.at[tt % 2]], wsem)
            return carry

        lax.fori_loop(0, _NTT, tt_body, 0, unroll=False)
        drain_scatter((_NTT - 1) % 2)

    return packing


_packing = _make_packing_kernel()


def kernel(ll, perm):
    lv = (ll.reshape(4, 128, _F, _C, _R)
            .transpose(2, 3, 0, 4, 1)
            .reshape(_NIN, _W))
    ov = _packing(lv, perm)
    out = (ov.reshape(_B, _C, _NTT, _R, 128)
             .transpose(0, 2, 4, 1, 3)
             .reshape(_B, _T, _C, _R))
    return out


# speedup vs baseline: 1.0010x; 1.0010x over previous
import functools

import jax
import jax.numpy as jnp
from jax import lax
from jax.experimental import pallas as pl
from jax.experimental.pallas import tpu as pltpu
from jax.experimental.pallas import tpu_sc as plsc

_B, _F, _C, _R = 512, 2000, 8, 4
_T = 2048
_L = 16
_NC, _NS = 2, 16
_NW = _NC * _NS

_NIN = _F * _C * 4
_NOUT = _B * _C * 16
_W = 512
_NTT = _T // 128
_QS = 32


def _make_packing_kernel(interpret=False):
    mesh = plsc.VectorSubcoreMesh(
        core_axis_name="c", subcore_axis_name="s",
        num_cores=_NC, num_subcores=_NS)

    @functools.partial(
        pl.kernel,
        out_type=jax.ShapeDtypeStruct((_NOUT, _W), jnp.float32),
        mesh=mesh,
        scratch_types=[
            pltpu.VMEM((_T,), jnp.int32),
            pltpu.VMEM((4, _QS), jnp.int32),
            pltpu.VMEM((2, 128), jnp.int32),
            pltpu.VMEM((2 * _QS + 1, _W), jnp.float32),
            pltpu.VMEM((128, _W), jnp.float32),
            pltpu.SemaphoreType.DMA,
            pltpu.SemaphoreType.DMA,
            pltpu.SemaphoreType.DMA,
        ],
        interpret=interpret,
        compiler_params=pltpu.CompilerParams(
            needs_layout_passes=False, use_tc_tiling_on_sc=False),
    )
    def packing(lv_hbm, perm_hbm, ov_hbm,
                perm_v, gidx_v, sidx_v, inb_v, outb_v, gsem0, gsem1, wsem):
        wid = lax.axis_index("s") * _NC + lax.axis_index("c")
        c = wid // 4
        bt = wid % 4
        pltpu.async_copy(perm_hbm, perm_v, wsem)

        lane = lax.iota(jnp.int32, _L)
        zeros = jnp.zeros((_L,), jnp.float32)
        gsems = (gsem0, gsem1)

        for q in range(_W // _L):
            inb_v[2 * _QS, pl.ds(q * _L, _L)] = zeros
        pltpu.make_async_copy(perm_hbm, perm_v, wsem).wait()

        obase = bt * 128 * _C * 16 + c * 16

        def build_gidx(tt, q, row):
            for u in range(_QS // _L):
                pv = perm_v[pl.ds(tt * 128 + q * _QS + u * _L, _L)]
                gidx_v[row, pl.ds(u * _L, _L)] = (
                    jnp.minimum(pv, _F - 1) * (_C * 4) + (c * 4 + bt))

        def fire_gather(row, half):
            pltpu.async_copy(
                lv_hbm.at[gidx_v.at[row]],
                inb_v.at[pl.ds(half * _QS, _QS)], gsems[half])

        def wait_gather(row, half):
            pltpu.make_async_copy(
                lv_hbm.at[gidx_v.at[row]],
                inb_v.at[pl.ds(half * _QS, _QS)], gsems[half]).wait()

        def drain_scatter(par):
            pltpu.make_async_copy(
                outb_v, ov_hbm.at[sidx_v.at[par]], wsem).wait()

        build_gidx(0, 0, 0)
        fire_gather(0, 0)

        def tt_body(tt, carry):
            for q in range(128 // _L):
                sidx_v[tt % 2, pl.ds(q * _L, _L)] = (
                    (lane + q * _L) * (_C * 16) + (obase + tt))

            for q in range(4):
                half = q % 2
                if q < 3:
                    build_gidx(tt, q + 1, q + 1)
                    fire_gather(q + 1, (q + 1) % 2)
                else:
                    @pl.when(tt != _NTT - 1)
                    def _():
                        build_gidx(tt + 1, 0, 0)
                        fire_gather(0, 0)

                wait_gather(q, half)

                if q == 0:
                    @pl.when(tt != 0)
                    def _():
                        drain_scatter((tt + 1) % 2)

                fi = []
                for tlc in range(_QS // _L):
                    pv = perm_v[pl.ds(tt * 128 + q * _QS + tlc * _L, _L)]
                    fi.append(jnp.where(pv < _F,
                                        lane + (half * _QS + tlc * _L),
                                        2 * _QS))

                ones = jnp.ones((_L,), jnp.int32)
                col0 = tuple(jnp.full((_L,), r * 128, jnp.int32)
                             for r in range(_R))

                @plsc.parallel_loop(0, 128, unroll=8, carry=col0)
                def j_body(j, cols):
                    for r in range(_R):
                        for tlc in range(_QS // _L):
                            v = plsc.load_gather(inb_v, [fi[tlc], cols[r]])
                            outb_v[j, pl.ds(r * 128 + q * _QS + tlc * _L,
                                            _L)] = v
                    return tuple(cv + ones for cv in cols)

            pltpu.async_copy(outb_v, ov_hbm.at[sidx_v.at[tt % 2]], wsem)
            return carry

        lax.fori_loop(0, _NTT, tt_body, 0, unroll=False)
        drain_scatter((_NTT - 1) % 2)

    return packing


_packing = _make_packing_kernel()


def kernel(ll, perm):
    lv = (ll.reshape(4, 128, _F, _C, _R)
            .transpose(2, 3, 0, 4, 1)
            .reshape(_NIN, _W))
    ov = _packing(lv, perm)
    out = (ov.reshape(_B, _C, _NTT, _R, 128)
             .transpose(0, 2, 4, 1, 3)
             .reshape(_B, _T, _C, _R))
    return out
